# baseline (speedup 1.0000x reference)
import functools

import jax
import jax.numpy as jnp
from jax import lax
from jax.experimental import pallas as pl
from jax.experimental.pallas import tpu as pltpu
from jax.experimental.pallas import tpu_sc as plsc

_B = 16384
_K = 32
_NW = 32
_BPW = _B // _NW
_GROUPS = _BPW // 16


def _bpr_body(users_hbm, items_hbm, negs_hbm, pt_hbm, qt_hbm,
              pos_hbm, neg_hbm,
              idx_u, idx_i, idx_n, buf_u, buf_i, buf_n,
              out_p, out_n, sem):
    cid = lax.axis_index("c")
    sid = lax.axis_index("s")
    wid = sid * 2 + cid
    base = wid * _BPW

    pltpu.sync_copy(users_hbm.at[pl.ds(base, _BPW)], idx_u)
    pltpu.sync_copy(items_hbm.at[pl.ds(base, _BPW)], idx_i)
    pltpu.sync_copy(negs_hbm.at[pl.ds(base, _BPW)], idx_n)

    copies = []
    for k in range(_K):
        copies.append(pltpu.async_copy(pt_hbm.at[k].at[idx_u], buf_u.at[k], sem))
        copies.append(pltpu.async_copy(qt_hbm.at[k].at[idx_i], buf_i.at[k], sem))
        copies.append(pltpu.async_copy(qt_hbm.at[k].at[idx_n], buf_n.at[k], sem))
    for cp in copies:
        cp.wait()

    zeros = jnp.zeros((16,), jnp.float32)

    def group(g, carry):
        off = g * 16
        acc_p = zeros
        acc_n = zeros
        for k in range(_K):
            u = buf_u[k, pl.ds(off, 16)]
            i = buf_i[k, pl.ds(off, 16)]
            n = buf_n[k, pl.ds(off, 16)]
            acc_p = acc_p + u * i
            acc_n = acc_n + u * n
        out_p[pl.ds(off, 16)] = acc_p
        out_n[pl.ds(off, 16)] = acc_n
        return carry

    lax.fori_loop(0, _GROUPS, group, 0)

    pltpu.sync_copy(out_p, pos_hbm.at[pl.ds(base, _BPW)])
    pltpu.sync_copy(out_n, neg_hbm.at[pl.ds(base, _BPW)])


@jax.jit
def _bpr(users, items, negs, Pt, Qt):
    mesh = plsc.VectorSubcoreMesh(core_axis_name="c", subcore_axis_name="s")
    run = functools.partial(
        pl.kernel,
        mesh=mesh,
        compiler_params=pltpu.CompilerParams(
            needs_layout_passes=False, use_tc_tiling_on_sc=False),
        out_type=(
            jax.ShapeDtypeStruct((_B,), jnp.float32),
            jax.ShapeDtypeStruct((_B,), jnp.float32),
        ),
        scratch_types=[
            pltpu.VMEM((_BPW,), jnp.int32),
            pltpu.VMEM((_BPW,), jnp.int32),
            pltpu.VMEM((_BPW,), jnp.int32),
            pltpu.VMEM((_K, _BPW), jnp.float32),
            pltpu.VMEM((_K, _BPW), jnp.float32),
            pltpu.VMEM((_K, _BPW), jnp.float32),
            pltpu.VMEM((_BPW,), jnp.float32),
            pltpu.VMEM((_BPW,), jnp.float32),
            pltpu.SemaphoreType.DMA,
        ],
    )(_bpr_body)
    return run(users, items, negs, Pt, Qt)


def kernel(users, items, neg_items, P, Q):
    pos, neg = _bpr(users.astype(jnp.int32), items.astype(jnp.int32),
                    neg_items.astype(jnp.int32), P.T, Q.T)
    return (pos, neg)
